# baseline (speedup 1.0000x reference)
import functools

import jax
import jax.numpy as jnp
from jax import lax
from jax.experimental import pallas as pl
from jax.experimental.pallas import tpu as pltpu
from jax.experimental.pallas import tpu_sc as plsc

N = 10000
E = 320000
IN = 128
HID = 16
HEADS = 8
OUT = 64
HC1 = HEADS * HID
HP = 16

NC = 2
NS = 16
NW = NC * NS
EPW = E // NW
CH1 = 40
NCHUNK1 = EPW // CH1
CH2 = 80
NCHUNK2 = EPW // CH2
ROWS_PER_STAGER = 1000

_f32 = jnp.float32


BR = 400


HH = HC1 // 2


def _dense1_body(x_ref, w_ref, asrc_ref, adst_ref, ha_ref, hb_ref,
                 as_ref, ad_ref):
    h = jnp.dot(x_ref[...], w_ref[...], preferred_element_type=_f32)
    ha_ref[...] = h[:, :HH]
    hb_ref[...] = h[:, HH:]
    as_ref[...] = jnp.dot(h, asrc_ref[...], preferred_element_type=_f32,
                          precision=lax.Precision.HIGHEST)
    ad_ref[...] = jnp.dot(h, adst_ref[...], preferred_element_type=_f32,
                          precision=lax.Precision.HIGHEST)


def _dense1(x, W1, A1s, A1d):
    return pl.pallas_call(
        _dense1_body,
        grid=(N // BR,),
        in_specs=[
            pl.BlockSpec((BR, IN), lambda i: (i, 0)),
            pl.BlockSpec((IN, HC1), lambda i: (0, 0)),
            pl.BlockSpec((HC1, HP), lambda i: (0, 0)),
            pl.BlockSpec((HC1, HP), lambda i: (0, 0)),
        ],
        out_specs=[
            pl.BlockSpec((BR, HH), lambda i: (i, 0)),
            pl.BlockSpec((BR, HH), lambda i: (i, 0)),
            pl.BlockSpec((BR, HP), lambda i: (i, 0)),
            pl.BlockSpec((BR, HP), lambda i: (i, 0)),
        ],
        out_shape=[
            jax.ShapeDtypeStruct((N, HH), _f32),
            jax.ShapeDtypeStruct((N, HH), _f32),
            jax.ShapeDtypeStruct((N, HP), _f32),
            jax.ShapeDtypeStruct((N, HP), _f32),
        ],
    )(x, W1, A1s, A1d)


def _dense2_body(pa0_ref, pa1_ref, pb0_ref, pb1_ref, d0_ref, d1_ref, exp_ref,
                 b1_ref, w2_ref, avs_ref, avd_ref, h2_ref, as_ref, ad_ref):
    recip = 1.0 / (d0_ref[...] + d1_ref[...] + 1e-16)
    rep = jnp.dot(recip, exp_ref[...], preferred_element_type=_f32,
                  precision=lax.Precision.HIGHEST)
    psum = jnp.concatenate(
        [pa0_ref[...] + pa1_ref[...], pb0_ref[...] + pb1_ref[...]], axis=1)
    x2 = jnp.maximum(psum * rep + b1_ref[...], 0.0)
    h2 = jnp.dot(x2, w2_ref[...], preferred_element_type=_f32)
    h2_ref[...] = h2
    as_ref[...] = jnp.dot(h2, avs_ref[...], preferred_element_type=_f32,
                          precision=lax.Precision.HIGHEST)
    ad_ref[...] = jnp.dot(h2, avd_ref[...], preferred_element_type=_f32,
                          precision=lax.Precision.HIGHEST)


def _dense2(pa0, pa1, pb0, pb1, d0, d1, EXPAND, b1, W2, avs, avd):
    return pl.pallas_call(
        _dense2_body,
        grid=(N // BR,),
        in_specs=[
            pl.BlockSpec((BR, HH), lambda i: (i, 0)),
            pl.BlockSpec((BR, HH), lambda i: (i, 0)),
            pl.BlockSpec((BR, HH), lambda i: (i, 0)),
            pl.BlockSpec((BR, HH), lambda i: (i, 0)),
            pl.BlockSpec((BR, HP), lambda i: (i, 0)),
            pl.BlockSpec((BR, HP), lambda i: (i, 0)),
            pl.BlockSpec((HP, HC1), lambda i: (0, 0)),
            pl.BlockSpec((1, HC1), lambda i: (0, 0)),
            pl.BlockSpec((HC1, OUT), lambda i: (0, 0)),
            pl.BlockSpec((OUT, 1), lambda i: (0, 0)),
            pl.BlockSpec((OUT, 1), lambda i: (0, 0)),
        ],
        out_specs=[
            pl.BlockSpec((BR, OUT), lambda i: (i, 0)),
            pl.BlockSpec((BR, 1), lambda i: (i, 0)),
            pl.BlockSpec((BR, 1), lambda i: (i, 0)),
        ],
        out_shape=[
            jax.ShapeDtypeStruct((N, OUT), _f32),
            jax.ShapeDtypeStruct((N, 1), _f32),
            jax.ShapeDtypeStruct((N, 1), _f32),
        ],
    )(pa0, pa1, pb0, pb1, d0, d1, EXPAND, b1, W2, avs, avd)


def _final_body(p0_ref, p1_ref, d0_ref, d1_ref, b2_ref, wfc_ref, bfc_ref,
                out_ref):
    recip = 1.0 / (d0_ref[...] + d1_ref[...] + 1e-16)
    y = (p0_ref[...] + p1_ref[...]) * recip + b2_ref[...]
    out_ref[...] = jnp.dot(y, wfc_ref[...], preferred_element_type=_f32) + bfc_ref[...]


def _final(p0, p1, d0, d1, b2, Wfc, bfc):
    return pl.pallas_call(
        _final_body,
        grid=(N // BR,),
        in_specs=[
            pl.BlockSpec((BR, OUT), lambda i: (i, 0)),
            pl.BlockSpec((BR, OUT), lambda i: (i, 0)),
            pl.BlockSpec((BR, 1), lambda i: (i, 0)),
            pl.BlockSpec((BR, 1), lambda i: (i, 0)),
            pl.BlockSpec((1, OUT), lambda i: (0, 0)),
            pl.BlockSpec((OUT, 2), lambda i: (0, 0)),
            pl.BlockSpec((1, 2), lambda i: (0, 0)),
        ],
        out_specs=pl.BlockSpec((BR, 2), lambda i: (i, 0)),
        out_shape=jax.ShapeDtypeStruct((N, 2), _f32),
    )(p0, p1, d0, d1, b2, Wfc, bfc)


_MESH = plsc.VectorSubcoreMesh(
    core_axis_name="c", subcore_axis_name="s", num_cores=NC, num_subcores=NS)


def _leaky_exp(t):
    return jnp.exp(jnp.maximum(t, 0.2 * t))


@functools.partial(
    pl.kernel,
    out_type=(
        jax.ShapeDtypeStruct((NC, N, HP), _f32),
        jax.ShapeDtypeStruct((NC, N, HH), _f32),
        jax.ShapeDtypeStruct((NC, N, HH), _f32),
    ),
    mesh=_MESH,
    compiler_params=pltpu.CompilerParams(use_tc_tiling_on_sc=False, needs_layout_passes=False),
    scratch_types=[
        pltpu.VMEM_SHARED((N, HP), _f32),
        pltpu.VMEM_SHARED((N, HH), _f32),
        pltpu.VMEM_SHARED((N, HH), _f32),
        pltpu.VMEM((EPW,), jnp.int32),
        pltpu.VMEM((EPW,), jnp.int32),
        pltpu.VMEM((CH1, HP), _f32),
        pltpu.VMEM((CH1, HP), _f32),
        pltpu.VMEM((CH1, HP), _f32),
        pltpu.VMEM((CH1, HP), _f32),
        pltpu.VMEM((CH1, HH), _f32),
        pltpu.VMEM((CH1, HH), _f32),
        pltpu.SemaphoreType.DMA,
        pltpu.SemaphoreType.DMA,
        pltpu.SemaphoreType.DMA,
        pltpu.SemaphoreType.DMA,
        pltpu.SemaphoreType.DMA,
        pltpu.SemaphoreType.DMA,
    ],
)
def _edge1(src_hbm, dst_hbm, ast_hbm, adt_hbm, ha_hbm, hb_hbm,
           z16_hbm, z64_hbm,
           dpart_hbm, oparta_hbm, opartb_hbm,
           den_sp, acc_sp, htab_sp, src_all, dst_all,
           s_v0, s_v1, d_v0, d_v1, h_v0, h_v1,
           semA0, semA1, semB0, semB1, semH0, semH1):
    c = lax.axis_index("c")
    s = lax.axis_index("s")
    wid = c * NS + s
    base = wid * EPW
    sv = (s_v0, s_v1)
    dv = (d_v0, d_v1)
    hv = (h_v0, h_v1)
    semA = (semA0, semA1)
    semB = (semB0, semB1)
    semH = (semH0, semH1)
    is_stager = s < N // ROWS_PER_STAGER
    rows = pl.ds(s * ROWS_PER_STAGER, ROWS_PER_STAGER)

    def _sweep(head_off, do_den):
        def _start(i, b):
            sl = pl.ds(i * CH1, CH1)
            pltpu.async_copy(ast_hbm.at[src_all.at[sl]], sv[b], semA[b])
            pltpu.async_copy(adt_hbm.at[dst_all.at[sl]], dv[b], semB[b])
            pltpu.async_copy(htab_sp.at[src_all.at[sl]], hv[b], semH[b])

        def _finish(i, b):
            sl = pl.ds(i * CH1, CH1)
            pltpu.make_async_copy(ast_hbm.at[src_all.at[sl]], sv[b], semA[b]).wait()
            pltpu.make_async_copy(adt_hbm.at[dst_all.at[sl]], dv[b], semB[b]).wait()
            pltpu.make_async_copy(htab_sp.at[src_all.at[sl]], hv[b], semH[b]).wait()

            @pl.loop(0, CH1)
            def _edge(j):
                ev = _leaky_exp(sv[b][j, :] + dv[b][j, :])
                if do_den:
                    sv[b][j, :] = ev
                for q in range(HEADS // 2):
                    cols = pl.ds(q * HID, HID)
                    hv[b][j, cols] = hv[b][j, cols] * ev[head_off + q]

            if do_den:
                pltpu.sync_copy(sv[b], den_sp.at[dst_all.at[sl]], add=True)
            pltpu.sync_copy(hv[b], acc_sp.at[dst_all.at[sl]], add=True)

        _start(0, 0)
        _start(1, 1)

        @pl.loop(0, (NCHUNK1 - 2) // 2)
        def _pair(k):
            i = k * 2
            _finish(i, 0)
            _start(i + 2, 0)
            _finish(i + 1, 1)

            @pl.when(i + 3 < NCHUNK1)
            def _():
                _start(i + 3, 1)

        _finish(NCHUNK1 - 2, 0)
        _finish(NCHUNK1 - 1, 1)

    @pl.when(is_stager)
    def _zero():
        pltpu.sync_copy(z16_hbm.at[rows], den_sp.at[rows])
        pltpu.sync_copy(z64_hbm.at[rows], acc_sp.at[rows])
        pltpu.sync_copy(ha_hbm.at[rows], htab_sp.at[rows])

    pltpu.sync_copy(src_hbm.at[pl.ds(base, EPW)], src_all)
    pltpu.sync_copy(dst_hbm.at[pl.ds(base, EPW)], dst_all)
    plsc.subcore_barrier()

    _sweep(0, True)
    plsc.subcore_barrier()

    @pl.when(is_stager)
    def _swap():
        pltpu.sync_copy(den_sp.at[rows], dpart_hbm.at[c, rows])
        pltpu.sync_copy(acc_sp.at[rows], oparta_hbm.at[c, rows])
        pltpu.sync_copy(z64_hbm.at[rows], acc_sp.at[rows])
        pltpu.sync_copy(hb_hbm.at[rows], htab_sp.at[rows])

    plsc.subcore_barrier()
    _sweep(HEADS // 2, False)
    plsc.subcore_barrier()

    @pl.when(is_stager)
    def _dump():
        pltpu.sync_copy(acc_sp.at[rows], opartb_hbm.at[c, rows])


@functools.partial(
    pl.kernel,
    out_type=(
        jax.ShapeDtypeStruct((NC, N), _f32),
        jax.ShapeDtypeStruct((NC, N, OUT), _f32),
    ),
    mesh=_MESH,
    compiler_params=pltpu.CompilerParams(use_tc_tiling_on_sc=False, needs_layout_passes=False),
    scratch_types=[
        pltpu.VMEM_SHARED((N,), _f32),
        pltpu.VMEM_SHARED((N, OUT), _f32),
        pltpu.VMEM_SHARED((N, OUT), _f32),
        pltpu.VMEM((N,), _f32),
        pltpu.VMEM((N,), _f32),
        pltpu.VMEM((CH2,), jnp.int32),
        pltpu.VMEM((CH2,), jnp.int32),
        pltpu.VMEM((CH2,), _f32),
        pltpu.VMEM((CH2, OUT), _f32),
        pltpu.SemaphoreType.DMA,
    ],
)
def _edge2(src_hbm, dst_hbm, ast_hbm, adt_hbm, h2_hbm, z1_hbm, z64_hbm,
           dpart_hbm, opart_hbm,
           den_sp, acc_sp, h2_sp, as_v, ad_v, src_v, dst_v, e_v, h_v, sem1):
    c = lax.axis_index("c")
    s = lax.axis_index("s")
    wid = c * NS + s

    pltpu.sync_copy(ast_hbm, as_v)
    pltpu.sync_copy(adt_hbm, ad_v)

    @pl.when(s < N // ROWS_PER_STAGER)
    def _zero():
        rows = pl.ds(s * ROWS_PER_STAGER, ROWS_PER_STAGER)
        pltpu.sync_copy(z1_hbm.at[rows], den_sp.at[rows])
        pltpu.sync_copy(z64_hbm.at[rows], acc_sp.at[rows])
        pltpu.sync_copy(h2_hbm.at[rows], h2_sp.at[rows])

    plsc.subcore_barrier()
    base = wid * EPW

    @pl.loop(0, NCHUNK2)
    def _chunk(i):
        off = base + i * CH2
        pltpu.sync_copy(src_hbm.at[pl.ds(off, CH2)], src_v)
        pltpu.sync_copy(dst_hbm.at[pl.ds(off, CH2)], dst_v)
        pltpu.async_copy(h2_sp.at[src_v], h_v, sem1)

        for k in range(CH2 // 16):
            sl = pl.ds(k * 16, 16)
            si = src_v[sl]
            di = dst_v[sl]
            svv = plsc.load_gather(as_v, [si])
            dvv = plsc.load_gather(ad_v, [di])
            e_v[sl] = _leaky_exp(svv + dvv)

        pltpu.sync_copy(e_v, den_sp.at[dst_v], add=True)
        pltpu.make_async_copy(h2_sp.at[src_v], h_v, sem1).wait()

        @pl.loop(0, CH2 // 16)
        def _scale(k):
            wv = e_v[pl.ds(k * 16, 16)]
            for jj in range(16):
                w = wv[jj]
                for q in range(OUT // 16):
                    cols = pl.ds(q * 16, 16)
                    h_v[k * 16 + jj, cols] = h_v[k * 16 + jj, cols] * w

        pltpu.sync_copy(h_v, acc_sp.at[dst_v], add=True)

    plsc.subcore_barrier()

    @pl.when(s < N // ROWS_PER_STAGER)
    def _dump():
        rows = pl.ds(s * ROWS_PER_STAGER, ROWS_PER_STAGER)
        pltpu.sync_copy(den_sp.at[rows], dpart_hbm.at[c, rows])
        pltpu.sync_copy(acc_sp.at[rows], opart_hbm.at[c, rows])


def kernel(x, edge_index, W1, a_src1, a_dst1, b1, W2, a_src2, a_dst2, b2,
           Wfc, bfc):
    src = edge_index[0]
    dst = edge_index[1]

    j = jnp.arange(HC1)
    A1s = jnp.zeros((HC1, HP), _f32).at[j, j // HID].set(a_src1.reshape(-1))
    A1d = jnp.zeros((HC1, HP), _f32).at[j, j // HID].set(a_dst1.reshape(-1))
    avs2 = a_src2.reshape(OUT, 1)
    avd2 = a_dst2.reshape(OUT, 1)
    EXP = jnp.zeros((HP, HC1), _f32).at[j // HID, j].set(1.0)

    z16 = jnp.zeros((N, HP), _f32)
    z64 = jnp.zeros((N, OUT), _f32)
    z1 = jnp.zeros((N,), _f32)

    ha, hb, as1, ad1 = _dense1(x, W1, A1s, A1d)
    dp1, opa, opb = _edge1(src, dst, as1, ad1, ha, hb, z16, z64)

    h2, as2, ad2 = _dense2(opa[0], opa[1], opb[0], opb[1], dp1[0], dp1[1], EXP,
                           b1.reshape(1, HC1), W2, avs2, avd2)
    dp2, op2 = _edge2(src, dst, as2.reshape(N), ad2.reshape(N), h2, z1, z64)

    return _final(op2[0], op2[1], dp2[0].reshape(N, 1), dp2[1].reshape(N, 1),
                  b2.reshape(1, OUT), Wfc, bfc.reshape(1, 2))


# speedup vs baseline: 1.3089x; 1.3089x over previous
import functools

import jax
import jax.numpy as jnp
from jax import lax
from jax.experimental import pallas as pl
from jax.experimental.pallas import tpu as pltpu
from jax.experimental.pallas import tpu_sc as plsc

N = 10000
E = 320000
IN = 128
HID = 16
HEADS = 8
OUT = 64
HC1 = HEADS * HID
HP = 16

NC = 2
NS = 16
NW = NC * NS
EPW = E // NW
CH1 = 40
NCHUNK1 = EPW // CH1
CH2 = 80
NCHUNK2 = EPW // CH2
ROWS_PER_STAGER = 1000

_f32 = jnp.float32


BR = 400


HX = HC1 + HP


def _dense1_body(x_ref, w_ref, asrc_ref, adst_ref, hx_ref, as_ref, ad_ref):
    h = jnp.dot(x_ref[...], w_ref[...], preferred_element_type=_f32)
    hx_ref[...] = jnp.concatenate([h, jnp.zeros((BR, HP), _f32)], axis=1)
    as_ref[...] = jnp.dot(h, asrc_ref[...], preferred_element_type=_f32,
                          precision=lax.Precision.HIGHEST)
    ad_ref[...] = jnp.dot(h, adst_ref[...], preferred_element_type=_f32,
                          precision=lax.Precision.HIGHEST)


def _dense1(x, W1, A1s, A1d):
    return pl.pallas_call(
        _dense1_body,
        grid=(N // BR,),
        in_specs=[
            pl.BlockSpec((BR, IN), lambda i: (i, 0)),
            pl.BlockSpec((IN, HC1), lambda i: (0, 0)),
            pl.BlockSpec((HC1, HP), lambda i: (0, 0)),
            pl.BlockSpec((HC1, HP), lambda i: (0, 0)),
        ],
        out_specs=[
            pl.BlockSpec((BR, HX), lambda i: (i, 0)),
            pl.BlockSpec((BR, HP), lambda i: (i, 0)),
            pl.BlockSpec((BR, HP), lambda i: (i, 0)),
        ],
        out_shape=[
            jax.ShapeDtypeStruct((N, HX), _f32),
            jax.ShapeDtypeStruct((N, HP), _f32),
            jax.ShapeDtypeStruct((N, HP), _f32),
        ],
    )(x, W1, A1s, A1d)


def _dense2_body(p0_ref, p1_ref, exp_ref, b1_ref, w2_ref, avs_ref, avd_ref,
                 h2_ref, as_ref, ad_ref):
    psum = p0_ref[...] + p1_ref[...]
    recip = 1.0 / (psum[:, HC1:] + 1e-16)
    rep = jnp.dot(recip, exp_ref[...], preferred_element_type=_f32,
                  precision=lax.Precision.HIGHEST)
    x2 = jnp.maximum(psum[:, :HC1] * rep + b1_ref[...], 0.0)
    h2 = jnp.dot(x2, w2_ref[...], preferred_element_type=_f32)
    h2_ref[...] = h2
    as_ref[...] = jnp.dot(h2, avs_ref[...], preferred_element_type=_f32,
                          precision=lax.Precision.HIGHEST)
    ad_ref[...] = jnp.dot(h2, avd_ref[...], preferred_element_type=_f32,
                          precision=lax.Precision.HIGHEST)


def _dense2(p0, p1, EXPAND, b1, W2, avs, avd):
    return pl.pallas_call(
        _dense2_body,
        grid=(N // BR,),
        in_specs=[
            pl.BlockSpec((BR, HX), lambda i: (i, 0)),
            pl.BlockSpec((BR, HX), lambda i: (i, 0)),
            pl.BlockSpec((HP, HC1), lambda i: (0, 0)),
            pl.BlockSpec((1, HC1), lambda i: (0, 0)),
            pl.BlockSpec((HC1, OUT), lambda i: (0, 0)),
            pl.BlockSpec((OUT, 1), lambda i: (0, 0)),
            pl.BlockSpec((OUT, 1), lambda i: (0, 0)),
        ],
        out_specs=[
            pl.BlockSpec((BR, OUT), lambda i: (i, 0)),
            pl.BlockSpec((BR, 1), lambda i: (i, 0)),
            pl.BlockSpec((BR, 1), lambda i: (i, 0)),
        ],
        out_shape=[
            jax.ShapeDtypeStruct((N, OUT), _f32),
            jax.ShapeDtypeStruct((N, 1), _f32),
            jax.ShapeDtypeStruct((N, 1), _f32),
        ],
    )(p0, p1, EXPAND, b1, W2, avs, avd)


def _final_body(p0_ref, p1_ref, d0_ref, d1_ref, b2_ref, wfc_ref, bfc_ref,
                out_ref):
    recip = 1.0 / (d0_ref[...] + d1_ref[...] + 1e-16)
    y = (p0_ref[...] + p1_ref[...]) * recip + b2_ref[...]
    out_ref[...] = jnp.dot(y, wfc_ref[...], preferred_element_type=_f32) + bfc_ref[...]


def _final(p0, p1, d0, d1, b2, Wfc, bfc):
    return pl.pallas_call(
        _final_body,
        grid=(N // BR,),
        in_specs=[
            pl.BlockSpec((BR, OUT), lambda i: (i, 0)),
            pl.BlockSpec((BR, OUT), lambda i: (i, 0)),
            pl.BlockSpec((BR, 1), lambda i: (i, 0)),
            pl.BlockSpec((BR, 1), lambda i: (i, 0)),
            pl.BlockSpec((1, OUT), lambda i: (0, 0)),
            pl.BlockSpec((OUT, 2), lambda i: (0, 0)),
            pl.BlockSpec((1, 2), lambda i: (0, 0)),
        ],
        out_specs=pl.BlockSpec((BR, 2), lambda i: (i, 0)),
        out_shape=jax.ShapeDtypeStruct((N, 2), _f32),
    )(p0, p1, d0, d1, b2, Wfc, bfc)


_MESH = plsc.VectorSubcoreMesh(
    core_axis_name="c", subcore_axis_name="s", num_cores=NC, num_subcores=NS)


def _leaky_exp(t):
    return jnp.exp(jnp.maximum(t, 0.2 * t))


@functools.partial(
    pl.kernel,
    out_type=jax.ShapeDtypeStruct((NC, N, HX), _f32),
    mesh=_MESH,
    compiler_params=pltpu.CompilerParams(use_tc_tiling_on_sc=False, needs_layout_passes=False),
    scratch_types=[
        pltpu.VMEM_SHARED((N, HX), _f32),
        pltpu.VMEM((EPW,), jnp.int32),
        pltpu.VMEM((EPW,), jnp.int32),
        pltpu.VMEM((CH1, HP), _f32),
        pltpu.VMEM((CH1, HP), _f32),
        pltpu.VMEM((CH1, HP), _f32),
        pltpu.VMEM((CH1, HP), _f32),
        pltpu.VMEM((CH1, HX), _f32),
        pltpu.VMEM((CH1, HX), _f32),
        pltpu.SemaphoreType.DMA,
        pltpu.SemaphoreType.DMA,
        pltpu.SemaphoreType.DMA,
        pltpu.SemaphoreType.DMA,
        pltpu.SemaphoreType.DMA,
        pltpu.SemaphoreType.DMA,
    ],
)
def _edge1(src_hbm, dst_hbm, ast_hbm, adt_hbm, hx_hbm, zhx_hbm,
           opart_hbm,
           acc_sp, src_all, dst_all,
           s_v0, s_v1, d_v0, d_v1, h_v0, h_v1,
           semA0, semA1, semB0, semB1, semH0, semH1):
    c = lax.axis_index("c")
    s = lax.axis_index("s")
    wid = c * NS + s
    base = wid * EPW
    sv = (s_v0, s_v1)
    dv = (d_v0, d_v1)
    hv = (h_v0, h_v1)
    semA = (semA0, semA1)
    semB = (semB0, semB1)
    semH = (semH0, semH1)

    @pl.when(s < N // ROWS_PER_STAGER)
    def _zero():
        rows = pl.ds(s * ROWS_PER_STAGER, ROWS_PER_STAGER)
        pltpu.sync_copy(zhx_hbm.at[rows], acc_sp.at[rows])

    pltpu.sync_copy(src_hbm.at[pl.ds(base, EPW)], src_all)
    pltpu.sync_copy(dst_hbm.at[pl.ds(base, EPW)], dst_all)
    plsc.subcore_barrier()

    def _start(i, b):
        sl = pl.ds(i * CH1, CH1)
        pltpu.async_copy(ast_hbm.at[src_all.at[sl]], sv[b], semA[b])
        pltpu.async_copy(adt_hbm.at[dst_all.at[sl]], dv[b], semB[b])
        pltpu.async_copy(hx_hbm.at[src_all.at[sl]], hv[b], semH[b])

    def _finish(i, b):
        sl = pl.ds(i * CH1, CH1)
        pltpu.make_async_copy(ast_hbm.at[src_all.at[sl]], sv[b], semA[b]).wait()
        pltpu.make_async_copy(adt_hbm.at[dst_all.at[sl]], dv[b], semB[b]).wait()
        pltpu.make_async_copy(hx_hbm.at[src_all.at[sl]], hv[b], semH[b]).wait()

        @pl.loop(0, CH1)
        def _edge(j):
            ev = _leaky_exp(sv[b][j, :] + dv[b][j, :])
            hv[b][j, pl.ds(HC1, HP)] = ev
            for h in range(HEADS):
                cols = pl.ds(h * HID, HID)
                hv[b][j, cols] = hv[b][j, cols] * ev[h]

        pltpu.sync_copy(hv[b], acc_sp.at[dst_all.at[sl]], add=True)

    _start(0, 0)
    _start(1, 1)

    @pl.loop(0, (NCHUNK1 - 2) // 2)
    def _pair(k):
        i = k * 2
        _finish(i, 0)
        _start(i + 2, 0)
        _finish(i + 1, 1)

        @pl.when(i + 3 < NCHUNK1)
        def _():
            _start(i + 3, 1)

    _finish(NCHUNK1 - 2, 0)
    _finish(NCHUNK1 - 1, 1)
    plsc.subcore_barrier()

    @pl.when(s < N // ROWS_PER_STAGER)
    def _dump():
        rows = pl.ds(s * ROWS_PER_STAGER, ROWS_PER_STAGER)
        pltpu.sync_copy(acc_sp.at[rows], opart_hbm.at[c, rows])


@functools.partial(
    pl.kernel,
    out_type=(
        jax.ShapeDtypeStruct((NC, N), _f32),
        jax.ShapeDtypeStruct((NC, N, OUT), _f32),
    ),
    mesh=_MESH,
    compiler_params=pltpu.CompilerParams(use_tc_tiling_on_sc=False, needs_layout_passes=False),
    scratch_types=[
        pltpu.VMEM_SHARED((N,), _f32),
        pltpu.VMEM_SHARED((N, OUT), _f32),
        pltpu.VMEM_SHARED((N, OUT), _f32),
        pltpu.VMEM((N,), _f32),
        pltpu.VMEM((N,), _f32),
        pltpu.VMEM((CH2,), jnp.int32),
        pltpu.VMEM((CH2,), jnp.int32),
        pltpu.VMEM((CH2,), _f32),
        pltpu.VMEM((CH2, OUT), _f32),
        pltpu.SemaphoreType.DMA,
    ],
)
def _edge2(src_hbm, dst_hbm, ast_hbm, adt_hbm, h2_hbm, z1_hbm, z64_hbm,
           dpart_hbm, opart_hbm,
           den_sp, acc_sp, h2_sp, as_v, ad_v, src_v, dst_v, e_v, h_v, sem1):
    c = lax.axis_index("c")
    s = lax.axis_index("s")
    wid = c * NS + s

    pltpu.sync_copy(ast_hbm, as_v)
    pltpu.sync_copy(adt_hbm, ad_v)

    @pl.when(s < N // ROWS_PER_STAGER)
    def _zero():
        rows = pl.ds(s * ROWS_PER_STAGER, ROWS_PER_STAGER)
        pltpu.sync_copy(z1_hbm.at[rows], den_sp.at[rows])
        pltpu.sync_copy(z64_hbm.at[rows], acc_sp.at[rows])
        pltpu.sync_copy(h2_hbm.at[rows], h2_sp.at[rows])

    plsc.subcore_barrier()
    base = wid * EPW

    @pl.loop(0, NCHUNK2)
    def _chunk(i):
        off = base + i * CH2
        pltpu.sync_copy(src_hbm.at[pl.ds(off, CH2)], src_v)
        pltpu.sync_copy(dst_hbm.at[pl.ds(off, CH2)], dst_v)
        pltpu.async_copy(h2_sp.at[src_v], h_v, sem1)

        for k in range(CH2 // 16):
            sl = pl.ds(k * 16, 16)
            si = src_v[sl]
            di = dst_v[sl]
            svv = plsc.load_gather(as_v, [si])
            dvv = plsc.load_gather(ad_v, [di])
            e_v[sl] = _leaky_exp(svv + dvv)

        pltpu.sync_copy(e_v, den_sp.at[dst_v], add=True)
        pltpu.make_async_copy(h2_sp.at[src_v], h_v, sem1).wait()

        @pl.loop(0, CH2 // 16)
        def _scale(k):
            wv = e_v[pl.ds(k * 16, 16)]
            for jj in range(16):
                w = wv[jj]
                for q in range(OUT // 16):
                    cols = pl.ds(q * 16, 16)
                    h_v[k * 16 + jj, cols] = h_v[k * 16 + jj, cols] * w

        pltpu.sync_copy(h_v, acc_sp.at[dst_v], add=True)

    plsc.subcore_barrier()

    @pl.when(s < N // ROWS_PER_STAGER)
    def _dump():
        rows = pl.ds(s * ROWS_PER_STAGER, ROWS_PER_STAGER)
        pltpu.sync_copy(den_sp.at[rows], dpart_hbm.at[c, rows])
        pltpu.sync_copy(acc_sp.at[rows], opart_hbm.at[c, rows])


def kernel(x, edge_index, W1, a_src1, a_dst1, b1, W2, a_src2, a_dst2, b2,
           Wfc, bfc):
    src = edge_index[0]
    dst = edge_index[1]

    j = jnp.arange(HC1)
    A1s = jnp.zeros((HC1, HP), _f32).at[j, j // HID].set(a_src1.reshape(-1))
    A1d = jnp.zeros((HC1, HP), _f32).at[j, j // HID].set(a_dst1.reshape(-1))
    avs2 = a_src2.reshape(OUT, 1)
    avd2 = a_dst2.reshape(OUT, 1)
    EXP = jnp.zeros((HP, HC1), _f32).at[j // HID, j].set(1.0)

    zhx = jnp.zeros((N, HX), _f32)
    z64 = jnp.zeros((N, OUT), _f32)
    z1 = jnp.zeros((N,), _f32)

    hx, as1, ad1 = _dense1(x, W1, A1s, A1d)
    op1 = _edge1(src, dst, as1, ad1, hx, zhx)

    h2, as2, ad2 = _dense2(op1[0], op1[1], EXP,
                           b1.reshape(1, HC1), W2, avs2, avd2)
    dp2, op2 = _edge2(src, dst, as2.reshape(N), ad2.reshape(N), h2, z1, z64)

    return _final(op2[0], op2[1], dp2[0].reshape(N, 1), dp2[1].reshape(N, 1),
                  b2.reshape(1, OUT), Wfc, bfc.reshape(1, 2))


# speedup vs baseline: 1.6041x; 1.2256x over previous
import functools

import jax
import jax.numpy as jnp
from jax import lax
from jax.experimental import pallas as pl
from jax.experimental.pallas import tpu as pltpu
from jax.experimental.pallas import tpu_sc as plsc

N = 10000
E = 320000
IN = 128
HID = 16
HEADS = 8
OUT = 64
HC1 = HEADS * HID
HP = 16

NC = 2
NS = 16
NW = NC * NS
EPW = E // NW
CH1 = 40
NCHUNK1 = EPW // CH1
CH2 = 80
NCHUNK2 = EPW // CH2
ROWS_PER_STAGER = 1000

_f32 = jnp.float32


BR = 400


HX = HC1 + HP
OX = OUT + HP


def _dense1_body(x_ref, w_ref, asrc_ref, adst_ref, hx_ref, as_ref, ad_ref):
    h = jnp.dot(x_ref[...], w_ref[...], preferred_element_type=_f32)
    hx_ref[...] = jnp.concatenate([h, jnp.zeros((BR, HP), _f32)], axis=1)
    as_ref[...] = jnp.dot(h, asrc_ref[...], preferred_element_type=_f32,
                          precision=lax.Precision.HIGHEST)
    ad_ref[...] = jnp.dot(h, adst_ref[...], preferred_element_type=_f32,
                          precision=lax.Precision.HIGHEST)


def _dense1(x, W1, A1s, A1d):
    return pl.pallas_call(
        _dense1_body,
        grid=(N // BR,),
        in_specs=[
            pl.BlockSpec((BR, IN), lambda i: (i, 0)),
            pl.BlockSpec((IN, HC1), lambda i: (0, 0)),
            pl.BlockSpec((HC1, HP), lambda i: (0, 0)),
            pl.BlockSpec((HC1, HP), lambda i: (0, 0)),
        ],
        out_specs=[
            pl.BlockSpec((BR, HX), lambda i: (i, 0)),
            pl.BlockSpec((BR, HP), lambda i: (i, 0)),
            pl.BlockSpec((BR, HP), lambda i: (i, 0)),
        ],
        out_shape=[
            jax.ShapeDtypeStruct((N, HX), _f32),
            jax.ShapeDtypeStruct((N, HP), _f32),
            jax.ShapeDtypeStruct((N, HP), _f32),
        ],
    )(x, W1, A1s, A1d)


def _dense2_body(p0_ref, p1_ref, exp_ref, b1_ref, w2_ref, avs_ref, avd_ref,
                 h2_ref, as_ref, ad_ref):
    psum = p0_ref[...] + p1_ref[...]
    recip = 1.0 / (psum[:, HC1:] + 1e-16)
    rep = jnp.dot(recip, exp_ref[...], preferred_element_type=_f32,
                  precision=lax.Precision.HIGHEST)
    x2 = jnp.maximum(psum[:, :HC1] * rep + b1_ref[...], 0.0)
    h2 = jnp.dot(x2, w2_ref[...], preferred_element_type=_f32)
    h2_ref[...] = jnp.concatenate([h2, jnp.ones((BR, HP), _f32)], axis=1)
    as_ref[...] = jnp.dot(h2, avs_ref[...], preferred_element_type=_f32,
                          precision=lax.Precision.HIGHEST)
    ad_ref[...] = jnp.dot(h2, avd_ref[...], preferred_element_type=_f32,
                          precision=lax.Precision.HIGHEST)


def _dense2(p0, p1, EXPAND, b1, W2, avs, avd):
    return pl.pallas_call(
        _dense2_body,
        grid=(N // BR,),
        in_specs=[
            pl.BlockSpec((BR, HX), lambda i: (i, 0)),
            pl.BlockSpec((BR, HX), lambda i: (i, 0)),
            pl.BlockSpec((HP, HC1), lambda i: (0, 0)),
            pl.BlockSpec((1, HC1), lambda i: (0, 0)),
            pl.BlockSpec((HC1, OUT), lambda i: (0, 0)),
            pl.BlockSpec((OUT, 1), lambda i: (0, 0)),
            pl.BlockSpec((OUT, 1), lambda i: (0, 0)),
        ],
        out_specs=[
            pl.BlockSpec((BR, OX), lambda i: (i, 0)),
            pl.BlockSpec((BR, 1), lambda i: (i, 0)),
            pl.BlockSpec((BR, 1), lambda i: (i, 0)),
        ],
        out_shape=[
            jax.ShapeDtypeStruct((N, OX), _f32),
            jax.ShapeDtypeStruct((N, 1), _f32),
            jax.ShapeDtypeStruct((N, 1), _f32),
        ],
    )(p0, p1, EXPAND, b1, W2, avs, avd)


def _final_body(p0_ref, p1_ref, b2_ref, wfc_ref, bfc_ref, out_ref):
    psum = p0_ref[...] + p1_ref[...]
    recip = 1.0 / (psum[:, OUT:OUT + 1] + 1e-16)
    y = psum[:, :OUT] * recip + b2_ref[...]
    out_ref[...] = jnp.dot(y, wfc_ref[...], preferred_element_type=_f32) + bfc_ref[...]


def _final(p0, p1, b2, Wfc, bfc):
    return pl.pallas_call(
        _final_body,
        grid=(N // BR,),
        in_specs=[
            pl.BlockSpec((BR, OX), lambda i: (i, 0)),
            pl.BlockSpec((BR, OX), lambda i: (i, 0)),
            pl.BlockSpec((1, OUT), lambda i: (0, 0)),
            pl.BlockSpec((OUT, 2), lambda i: (0, 0)),
            pl.BlockSpec((1, 2), lambda i: (0, 0)),
        ],
        out_specs=pl.BlockSpec((BR, 2), lambda i: (i, 0)),
        out_shape=jax.ShapeDtypeStruct((N, 2), _f32),
    )(p0, p1, b2, Wfc, bfc)


_MESH = plsc.VectorSubcoreMesh(
    core_axis_name="c", subcore_axis_name="s", num_cores=NC, num_subcores=NS)


def _leaky_exp(t):
    return jnp.exp(jnp.maximum(t, 0.2 * t))


@functools.partial(
    pl.kernel,
    out_type=jax.ShapeDtypeStruct((NC, N, HX), _f32),
    mesh=_MESH,
    compiler_params=pltpu.CompilerParams(use_tc_tiling_on_sc=False, needs_layout_passes=False),
    scratch_types=[
        pltpu.VMEM_SHARED((N, HX), _f32),
        pltpu.VMEM((EPW,), jnp.int32),
        pltpu.VMEM((EPW,), jnp.int32),
        pltpu.VMEM((CH1, HP), _f32),
        pltpu.VMEM((CH1, HP), _f32),
        pltpu.VMEM((CH1, HP), _f32),
        pltpu.VMEM((CH1, HP), _f32),
        pltpu.VMEM((CH1, HX), _f32),
        pltpu.VMEM((CH1, HX), _f32),
        pltpu.SemaphoreType.DMA,
        pltpu.SemaphoreType.DMA,
        pltpu.SemaphoreType.DMA,
        pltpu.SemaphoreType.DMA,
        pltpu.SemaphoreType.DMA,
        pltpu.SemaphoreType.DMA,
    ],
)
def _edge1(src_hbm, dst_hbm, ast_hbm, adt_hbm, hx_hbm, zhx_hbm,
           opart_hbm,
           acc_sp, src_all, dst_all,
           s_v0, s_v1, d_v0, d_v1, h_v0, h_v1,
           semA0, semA1, semB0, semB1, semH0, semH1):
    c = lax.axis_index("c")
    s = lax.axis_index("s")
    wid = c * NS + s
    base = wid * EPW
    sv = (s_v0, s_v1)
    dv = (d_v0, d_v1)
    hv = (h_v0, h_v1)
    semA = (semA0, semA1)
    semB = (semB0, semB1)
    semH = (semH0, semH1)

    @pl.when(s < N // ROWS_PER_STAGER)
    def _zero():
        rows = pl.ds(s * ROWS_PER_STAGER, ROWS_PER_STAGER)
        pltpu.sync_copy(zhx_hbm.at[rows], acc_sp.at[rows])

    pltpu.sync_copy(src_hbm.at[pl.ds(base, EPW)], src_all)
    pltpu.sync_copy(dst_hbm.at[pl.ds(base, EPW)], dst_all)
    plsc.subcore_barrier()

    def _start(i, b):
        sl = pl.ds(i * CH1, CH1)
        pltpu.async_copy(ast_hbm.at[src_all.at[sl]], sv[b], semA[b])
        pltpu.async_copy(adt_hbm.at[dst_all.at[sl]], dv[b], semB[b])
        pltpu.async_copy(hx_hbm.at[src_all.at[sl]], hv[b], semH[b])

    def _finish(i, b):
        sl = pl.ds(i * CH1, CH1)
        pltpu.make_async_copy(ast_hbm.at[src_all.at[sl]], sv[b], semA[b]).wait()
        pltpu.make_async_copy(adt_hbm.at[dst_all.at[sl]], dv[b], semB[b]).wait()
        pltpu.make_async_copy(hx_hbm.at[src_all.at[sl]], hv[b], semH[b]).wait()

        @pl.loop(0, CH1)
        def _edge(j):
            ev = _leaky_exp(sv[b][j, :] + dv[b][j, :])
            hv[b][j, pl.ds(HC1, HP)] = ev
            for h in range(HEADS):
                cols = pl.ds(h * HID, HID)
                hv[b][j, cols] = hv[b][j, cols] * ev[h]

        pltpu.sync_copy(hv[b], acc_sp.at[dst_all.at[sl]], add=True)

    _start(0, 0)
    _start(1, 1)

    @pl.loop(0, (NCHUNK1 - 2) // 2)
    def _pair(k):
        i = k * 2
        _finish(i, 0)
        _start(i + 2, 0)
        _finish(i + 1, 1)

        @pl.when(i + 3 < NCHUNK1)
        def _():
            _start(i + 3, 1)

    _finish(NCHUNK1 - 2, 0)
    _finish(NCHUNK1 - 1, 1)
    plsc.subcore_barrier()

    @pl.when(s < N // ROWS_PER_STAGER)
    def _dump():
        rows = pl.ds(s * ROWS_PER_STAGER, ROWS_PER_STAGER)
        pltpu.sync_copy(acc_sp.at[rows], opart_hbm.at[c, rows])


@functools.partial(
    pl.kernel,
    out_type=jax.ShapeDtypeStruct((NC, N, OX), _f32),
    mesh=_MESH,
    compiler_params=pltpu.CompilerParams(use_tc_tiling_on_sc=False, needs_layout_passes=False),
    scratch_types=[
        pltpu.VMEM_SHARED((N, OX), _f32),
        pltpu.VMEM((N,), _f32),
        pltpu.VMEM((N,), _f32),
        pltpu.VMEM((EPW,), jnp.int32),
        pltpu.VMEM((EPW,), jnp.int32),
        pltpu.VMEM((CH2,), _f32),
        pltpu.VMEM((CH2,), _f32),
        pltpu.VMEM((CH2, OX), _f32),
        pltpu.VMEM((CH2, OX), _f32),
        pltpu.SemaphoreType.DMA,
        pltpu.SemaphoreType.DMA,
    ],
)
def _edge2(src_hbm, dst_hbm, ast_hbm, adt_hbm, h2x_hbm, zox_hbm,
           opart_hbm,
           acc_sp, as_v, ad_v, src_all, dst_all,
           e_v0, e_v1, h_v0, h_v1, semH0, semH1):
    c = lax.axis_index("c")
    s = lax.axis_index("s")
    wid = c * NS + s
    base = wid * EPW
    ev = (e_v0, e_v1)
    hv = (h_v0, h_v1)
    semH = (semH0, semH1)

    pltpu.sync_copy(ast_hbm, as_v)
    pltpu.sync_copy(adt_hbm, ad_v)

    @pl.when(s < N // ROWS_PER_STAGER)
    def _zero():
        rows = pl.ds(s * ROWS_PER_STAGER, ROWS_PER_STAGER)
        pltpu.sync_copy(zox_hbm.at[rows], acc_sp.at[rows])

    pltpu.sync_copy(src_hbm.at[pl.ds(base, EPW)], src_all)
    pltpu.sync_copy(dst_hbm.at[pl.ds(base, EPW)], dst_all)
    plsc.subcore_barrier()

    def _start(i, b):
        sl = pl.ds(i * CH2, CH2)
        pltpu.async_copy(h2x_hbm.at[src_all.at[sl]], hv[b], semH[b])

    def _finish(i, b):
        sl = pl.ds(i * CH2, CH2)

        for k in range(CH2 // 16):
            sl16 = pl.ds(i * CH2 + k * 16, 16)
            si = src_all[sl16]
            di = dst_all[sl16]
            svv = plsc.load_gather(as_v, [si])
            dvv = plsc.load_gather(ad_v, [di])
            ev[b][pl.ds(k * 16, 16)] = _leaky_exp(svv + dvv)

        pltpu.make_async_copy(h2x_hbm.at[src_all.at[sl]], hv[b], semH[b]).wait()

        @pl.loop(0, CH2 // 16)
        def _scale(k):
            wv = ev[b][pl.ds(k * 16, 16)]
            for jj in range(16):
                w = wv[jj]
                for q in range(OX // 16):
                    cols = pl.ds(q * 16, 16)
                    h_v = hv[b]
                    h_v[k * 16 + jj, cols] = h_v[k * 16 + jj, cols] * w

        pltpu.sync_copy(hv[b], acc_sp.at[dst_all.at[sl]], add=True)

    _start(0, 0)
    _start(1, 1)

    @pl.loop(0, (NCHUNK2 - 1) // 2)
    def _pair(k):
        i = k * 2
        _finish(i, 0)
        _start(i + 2, 0)
        _finish(i + 1, 1)

        @pl.when(i + 3 < NCHUNK2)
        def _():
            _start(i + 3, 1)

    _finish(NCHUNK2 - 1, 0)
    plsc.subcore_barrier()

    @pl.when(s < N // ROWS_PER_STAGER)
    def _dump():
        rows = pl.ds(s * ROWS_PER_STAGER, ROWS_PER_STAGER)
        pltpu.sync_copy(acc_sp.at[rows], opart_hbm.at[c, rows])


def kernel(x, edge_index, W1, a_src1, a_dst1, b1, W2, a_src2, a_dst2, b2,
           Wfc, bfc):
    src = edge_index[0]
    dst = edge_index[1]

    j = jnp.arange(HC1)
    A1s = jnp.zeros((HC1, HP), _f32).at[j, j // HID].set(a_src1.reshape(-1))
    A1d = jnp.zeros((HC1, HP), _f32).at[j, j // HID].set(a_dst1.reshape(-1))
    avs2 = a_src2.reshape(OUT, 1)
    avd2 = a_dst2.reshape(OUT, 1)
    EXP = jnp.zeros((HP, HC1), _f32).at[j // HID, j].set(1.0)

    zhx = jnp.zeros((N, HX), _f32)
    zox = jnp.zeros((N, OX), _f32)

    hx, as1, ad1 = _dense1(x, W1, A1s, A1d)
    op1 = _edge1(src, dst, as1, ad1, hx, zhx)

    h2x, as2, ad2 = _dense2(op1[0], op1[1], EXP,
                            b1.reshape(1, HC1), W2, avs2, avd2)
    op2 = _edge2(src, dst, as2.reshape(N), ad2.reshape(N), h2x, zox)

    return _final(op2[0], op2[1], b2.reshape(1, OUT), Wfc, bfc.reshape(1, 2))
